# Initial kernel scaffold; baseline (speedup 1.0000x reference)
import jax
import jax.numpy as jnp
from jax.experimental import pallas as pl


def kernel(output, labels, images, reconstructions):
    raise NotImplementedError("write your pallas kernel here")


# speedup vs baseline: 1.0970x; 1.0970x over previous
import jax
import jax.numpy as jnp
from jax.experimental import pallas as pl

_GAMMA = 2.0
_ALPHA = 0.5
_NUM_HARD = 2


def _focal_body(x0, x1, x2, x3, x4, l0, l1, l2, l3, l4, out_ref):
    cls = l0[...]
    x = x0[...]
    pos = cls > 0.5
    neg = cls < -0.5
    pos_count = jnp.sum(jnp.where(pos, 1.0, 0.0))
    neg_count = jnp.sum(jnp.where(neg, 1.0, 0.0))

    p = jax.nn.sigmoid(x)
    fp = -((1.0 - p) * (1.0 - p)) * (jnp.log(p) * 0.5)
    focal_pos = jnp.sum(jnp.where(pos, fp, 0.0))
    pos_correct = jnp.sum(jnp.where(pos & (p >= 0.5), 1.0, 0.0))

    denom = jnp.maximum(pos_count, 1.0)
    rl = []
    for xr, lr in ((x1, l1), (x2, l2), (x3, l3), (x4, l4)):
        d = xr[...] - lr[...]
        ad = jnp.abs(d)
        sl = jnp.where(ad < 1.0, 0.5 * d * d, ad - 0.5)
        s = jnp.sum(jnp.where(pos, sl, 0.0))
        rl.append(jnp.where(pos_count > 0, s / denom, 0.0))

    neg_inf = jnp.float32(-jnp.inf)
    s = jnp.where(neg, x, neg_inf)
    ri = jax.lax.broadcasted_iota(jnp.int32, s.shape, 0)
    big = jnp.int32(2 ** 30)
    cands = []
    for _ in range(16):
        m = jnp.max(s, axis=0, keepdims=True)
        sel = s == m
        idx = jnp.min(jnp.where(sel, ri, big), axis=0, keepdims=True)
        s = jnp.where(sel & (ri == idx), neg_inf, s)
        cands.append(m)
    cand = jnp.concatenate(cands, axis=0)

    fi = (jax.lax.broadcasted_iota(jnp.int32, cand.shape, 0) * 128
          + jax.lax.broadcasted_iota(jnp.int32, cand.shape, 1))
    focal_neg = jnp.float32(0.0)
    neg_correct = jnp.float32(0.0)
    g = cand
    for _ in range(16):
        gm = jnp.max(g)
        sel = g == gm
        idx = jnp.min(jnp.where(sel, fi, big))
        g = jnp.where(sel & (fi == idx), neg_inf, g)
        valid = gm > neg_inf
        pn = jax.nn.sigmoid(gm)
        ptn = 1.0 - pn
        ln = -(pn * pn) * (jnp.log(ptn) * 0.5)
        focal_neg = focal_neg + jnp.where(valid, ln, 0.0)
        neg_correct = neg_correct + jnp.where(valid & (pn < 0.5), 1.0, 0.0)

    neg_k = jnp.minimum(neg_count, 16.0)
    classify = (focal_pos + focal_neg) / (pos_count + neg_k)

    vals = (classify, rl[0], rl[1], rl[2], rl[3],
            pos_count, pos_correct, neg_correct, neg_k)
    li = jax.lax.broadcasted_iota(jnp.int32, (1, 128), 1)
    acc = jnp.zeros((1, 128), jnp.float32)
    for slot, v in enumerate(vals):
        acc = jnp.where(li == slot, v, acc)
    out_ref[...] = acc


def _recon_body(img_ref, rec_ref, out_ref):
    i = pl.program_id(0)

    @pl.when(i == 0)
    def _():
        out_ref[...] = jnp.zeros((1, 1), jnp.float32)

    d = rec_ref[...] - img_ref[...]
    out_ref[...] += jnp.sum(d * d).reshape(1, 1)


@jax.jit
def kernel(output, labels, images, reconstructions):
    out5 = output.reshape(-1, 5)
    lab5 = labels.reshape(-1, 5)
    n = out5.shape[0]
    rows = n // 128
    cols = [out5[:, i].reshape(rows, 128) for i in range(5)]
    lcols = [lab5[:, i].reshape(rows, 128) for i in range(5)]

    res = pl.pallas_call(
        _focal_body,
        out_shape=jax.ShapeDtypeStruct((1, 128), jnp.float32),
    )(*cols, *lcols)

    m = images.size
    lanes = 1024
    r = m // lanes
    steps = 8
    blk = r // steps
    img2 = images.reshape(r, lanes)
    rec2 = reconstructions.reshape(r, lanes)
    sq = pl.pallas_call(
        _recon_body,
        grid=(steps,),
        in_specs=[
            pl.BlockSpec((blk, lanes), lambda i: (i, 0)),
            pl.BlockSpec((blk, lanes), lambda i: (i, 0)),
        ],
        out_specs=pl.BlockSpec((1, 1), lambda i: (0, 0)),
        out_shape=jax.ShapeDtypeStruct((1, 1), jnp.float32),
    )(img2, rec2)

    recon_loss = 1e-06 * (sq[0, 0] / jnp.float32(m))
    classify_loss = res[0, 0]
    rl = [res[0, 1], res[0, 2], res[0, 3], res[0, 4]]
    loss = classify_loss + rl[0] + rl[1] + rl[2] + rl[3] + recon_loss
    pos_correct = res[0, 6].astype(jnp.int32)
    pos_total = res[0, 5].astype(jnp.int32)
    neg_correct = res[0, 7].astype(jnp.int32)
    neg_total = res[0, 8].astype(jnp.int32)
    return (loss, classify_loss, rl[0], rl[1], rl[2], rl[3],
            pos_correct, pos_total, neg_correct, neg_total, recon_loss)


# speedup vs baseline: 1.1628x; 1.0600x over previous
import jax
import jax.numpy as jnp
from jax.experimental import pallas as pl
from jax.experimental.pallas import tpu as pltpu

_LANES = 640


def _focal_body(o_ref, l_ref, out_ref):
    out = o_ref[...]
    lab = l_ref[...]
    shape = out.shape

    ch = jax.lax.broadcasted_iota(jnp.int32, shape, 1) % 5
    m0 = ch == 0
    pos0 = m0 & (lab > 0.5)
    neg0 = m0 & (lab < -0.5)
    pos_count = jnp.sum(jnp.where(pos0, 1.0, 0.0))
    neg_count = jnp.sum(jnp.where(neg0, 1.0, 0.0))

    p = jax.nn.sigmoid(out)
    fp = -((1.0 - p) * (1.0 - p)) * (jnp.log(p) * 0.5)
    focal_pos = jnp.sum(jnp.where(pos0, fp, 0.0))
    pos_correct = jnp.sum(jnp.where(pos0 & (p >= 0.5), 1.0, 0.0))

    pf = jnp.where(pos0, 1.0, 0.0)
    r = pf
    for _ in range(4):
        r = pltpu.roll(r, 1, 1)
        pf = pf + r
    posm = pf > 0.5

    d = out - lab
    ad = jnp.abs(d)
    sl = jnp.where(ad < 1.0, 0.5 * d * d, ad - 0.5)
    slp = jnp.where(posm, sl, 0.0)
    denom = jnp.maximum(pos_count, 1.0)
    rl = []
    for c in range(1, 5):
        s = jnp.sum(jnp.where(ch == c, slp, 0.0))
        rl.append(jnp.where(pos_count > 0, s / denom, 0.0))

    neg_inf = jnp.float32(-jnp.inf)
    s = jnp.where(neg0, out, neg_inf)
    ri = jax.lax.broadcasted_iota(jnp.int32, shape, 0)
    big = jnp.int32(2 ** 30)
    cands = []
    for _ in range(16):
        m = jnp.max(s, axis=0, keepdims=True)
        sel = s == m
        idx = jnp.min(jnp.where(sel, ri, big), axis=0, keepdims=True)
        s = jnp.where(sel & (ri == idx), neg_inf, s)
        cands.append(m)
    cand = jnp.concatenate(cands, axis=0)

    fi = (jax.lax.broadcasted_iota(jnp.int32, cand.shape, 0) * _LANES
          + jax.lax.broadcasted_iota(jnp.int32, cand.shape, 1))
    focal_neg = jnp.float32(0.0)
    neg_correct = jnp.float32(0.0)
    g = cand
    for _ in range(16):
        gm = jnp.max(g)
        sel = g == gm
        idx = jnp.min(jnp.where(sel, fi, big))
        g = jnp.where(sel & (fi == idx), neg_inf, g)
        valid = gm > neg_inf
        pn = jax.nn.sigmoid(gm)
        ptn = 1.0 - pn
        ln = -(pn * pn) * (jnp.log(ptn) * 0.5)
        focal_neg = focal_neg + jnp.where(valid, ln, 0.0)
        neg_correct = neg_correct + jnp.where(valid & (pn < 0.5), 1.0, 0.0)

    neg_k = jnp.minimum(neg_count, 16.0)
    classify = (focal_pos + focal_neg) / (pos_count + neg_k)

    vals = (classify, rl[0], rl[1], rl[2], rl[3],
            pos_count, pos_correct, neg_correct, neg_k)
    li = jax.lax.broadcasted_iota(jnp.int32, (1, 128), 1)
    acc = jnp.zeros((1, 128), jnp.float32)
    for slot, v in enumerate(vals):
        acc = jnp.where(li == slot, v, acc)
    out_ref[...] = acc


def _recon_body(img_ref, rec_ref, out_ref):
    i = pl.program_id(0)

    @pl.when(i == 0)
    def _():
        out_ref[...] = jnp.zeros((1, 1), jnp.float32)

    d = rec_ref[...] - img_ref[...]
    out_ref[...] += jnp.sum(d * d).reshape(1, 1)


@jax.jit
def kernel(output, labels, images, reconstructions):
    n = output.size // 5
    rows = n // 128
    o2 = output.reshape(rows, _LANES)
    l2 = labels.reshape(rows, _LANES)

    res = pl.pallas_call(
        _focal_body,
        out_shape=jax.ShapeDtypeStruct((1, 128), jnp.float32),
    )(o2, l2)

    m = images.size
    lanes = 1024
    r = m // lanes
    steps = 8
    blk = r // steps
    img2 = images.reshape(r, lanes)
    rec2 = reconstructions.reshape(r, lanes)
    sq = pl.pallas_call(
        _recon_body,
        grid=(steps,),
        in_specs=[
            pl.BlockSpec((blk, lanes), lambda i: (i, 0)),
            pl.BlockSpec((blk, lanes), lambda i: (i, 0)),
        ],
        out_specs=pl.BlockSpec((1, 1), lambda i: (0, 0)),
        out_shape=jax.ShapeDtypeStruct((1, 1), jnp.float32),
    )(img2, rec2)

    recon_loss = 1e-06 * (sq[0, 0] / jnp.float32(m))
    classify_loss = res[0, 0]
    rl = [res[0, 1], res[0, 2], res[0, 3], res[0, 4]]
    loss = classify_loss + rl[0] + rl[1] + rl[2] + rl[3] + recon_loss
    pos_correct = res[0, 6].astype(jnp.int32)
    pos_total = res[0, 5].astype(jnp.int32)
    neg_correct = res[0, 7].astype(jnp.int32)
    neg_total = res[0, 8].astype(jnp.int32)
    return (loss, classify_loss, rl[0], rl[1], rl[2], rl[3],
            pos_correct, pos_total, neg_correct, neg_total, recon_loss)


# speedup vs baseline: 1.1695x; 1.0058x over previous
import jax
import jax.numpy as jnp
from jax.experimental import pallas as pl
from jax.experimental.pallas import tpu as pltpu

_LANES = 640


def _focal_body(o_ref, l_ref, out_ref):
    out = o_ref[...]
    lab = l_ref[...]
    shape = out.shape

    ch = jax.lax.broadcasted_iota(jnp.int32, shape, 1) % 5
    m0 = ch == 0
    pos0 = m0 & (lab > 0.5)
    neg0 = m0 & (lab < -0.5)
    pos_count = jnp.sum(jnp.where(pos0, 1.0, 0.0))
    neg_count = jnp.sum(jnp.where(neg0, 1.0, 0.0))

    p = jax.nn.sigmoid(out)
    fp = -((1.0 - p) * (1.0 - p)) * (jnp.log(p) * 0.5)
    focal_pos = jnp.sum(jnp.where(pos0, fp, 0.0))
    pos_correct = jnp.sum(jnp.where(pos0 & (p >= 0.5), 1.0, 0.0))

    pf = jnp.where(pos0, 1.0, 0.0)
    r = pf
    for _ in range(4):
        r = pltpu.roll(r, 1, 1)
        pf = pf + r
    posm = pf > 0.5

    d = out - lab
    ad = jnp.abs(d)
    sl = jnp.where(ad < 1.0, 0.5 * d * d, ad - 0.5)
    slp = jnp.where(posm, sl, 0.0)
    denom = jnp.maximum(pos_count, 1.0)
    rl = []
    for c in range(1, 5):
        s = jnp.sum(jnp.where(ch == c, slp, 0.0))
        rl.append(jnp.where(pos_count > 0, s / denom, 0.0))

    neg_inf = jnp.float32(-jnp.inf)
    s = jnp.where(neg0, out, neg_inf)
    ri = jax.lax.broadcasted_iota(jnp.int32, shape, 0)
    big = jnp.int32(2 ** 30)
    cands = []
    for _ in range(16):
        m = jnp.max(s, axis=0, keepdims=True)
        sel = s == m
        idx = jnp.min(jnp.where(sel, ri, big), axis=0, keepdims=True)
        s = jnp.where(sel & (ri == idx), neg_inf, s)
        cands.append(m)
    cand = jnp.concatenate(cands, axis=0)

    fi = (jax.lax.broadcasted_iota(jnp.int32, cand.shape, 0) * _LANES
          + jax.lax.broadcasted_iota(jnp.int32, cand.shape, 1))
    focal_neg = jnp.float32(0.0)
    neg_correct = jnp.float32(0.0)
    g = cand
    for _ in range(16):
        gm = jnp.max(g)
        sel = g == gm
        idx = jnp.min(jnp.where(sel, fi, big))
        g = jnp.where(sel & (fi == idx), neg_inf, g)
        valid = gm > neg_inf
        pn = jax.nn.sigmoid(gm)
        ptn = 1.0 - pn
        ln = -(pn * pn) * (jnp.log(ptn) * 0.5)
        focal_neg = focal_neg + jnp.where(valid, ln, 0.0)
        neg_correct = neg_correct + jnp.where(valid & (pn < 0.5), 1.0, 0.0)

    neg_k = jnp.minimum(neg_count, 16.0)
    classify = (focal_pos + focal_neg) / (pos_count + neg_k)

    vals = (classify, rl[0], rl[1], rl[2], rl[3],
            pos_count, pos_correct, neg_correct, neg_k)
    li = jax.lax.broadcasted_iota(jnp.int32, (1, 128), 1)
    acc = jnp.zeros((1, 128), jnp.float32)
    for slot, v in enumerate(vals):
        acc = jnp.where(li == slot, v, acc)
    out_ref[...] = acc


def _recon_body(img_ref, rec_ref, out_ref):
    i = pl.program_id(0)

    @pl.when(i == 0)
    def _():
        out_ref[...] = jnp.zeros((1, 1), jnp.float32)

    d = rec_ref[...] - img_ref[...]
    out_ref[...] += jnp.sum(d * d).reshape(1, 1)


@jax.jit
def kernel(output, labels, images, reconstructions):
    n = output.size // 5
    rows = n // 128
    o2 = output.reshape(rows, _LANES)
    l2 = labels.reshape(rows, _LANES)

    res = pl.pallas_call(
        _focal_body,
        out_shape=jax.ShapeDtypeStruct((1, 128), jnp.float32),
    )(o2, l2)

    m = images.size
    v = images.shape[-1]
    r = m // (v * v)
    steps = 8
    blk = r // steps
    img2 = images.reshape(r, v, v)
    rec2 = reconstructions.reshape(r, v, v)
    sq = pl.pallas_call(
        _recon_body,
        grid=(steps,),
        in_specs=[
            pl.BlockSpec((blk, v, v), lambda i: (i, 0, 0)),
            pl.BlockSpec((blk, v, v), lambda i: (i, 0, 0)),
        ],
        out_specs=pl.BlockSpec((1, 1), lambda i: (0, 0)),
        out_shape=jax.ShapeDtypeStruct((1, 1), jnp.float32),
    )(img2, rec2)

    recon_loss = 1e-06 * (sq[0, 0] / jnp.float32(m))
    classify_loss = res[0, 0]
    rl = [res[0, 1], res[0, 2], res[0, 3], res[0, 4]]
    loss = classify_loss + rl[0] + rl[1] + rl[2] + rl[3] + recon_loss
    pos_correct = res[0, 6].astype(jnp.int32)
    pos_total = res[0, 5].astype(jnp.int32)
    neg_correct = res[0, 7].astype(jnp.int32)
    neg_total = res[0, 8].astype(jnp.int32)
    return (loss, classify_loss, rl[0], rl[1], rl[2], rl[3],
            pos_correct, pos_total, neg_correct, neg_total, recon_loss)
